# Initial kernel scaffold; baseline (speedup 1.0000x reference)
import jax
import jax.numpy as jnp
from jax.experimental import pallas as pl


def kernel(x, edge_index, edge_type, batch, W1, root1, b1, W2, root2, b2, linW, linb):
    raise NotImplementedError("write your pallas kernel here")


# speedup vs baseline: 4.4959x; 4.4959x over previous
import functools

import jax
import jax.numpy as jnp
from jax import lax
from jax.experimental import pallas as pl
from jax.experimental.pallas import tpu as pltpu
from jax.experimental.pallas import tpu_sc as plsc

N_ = 100000
E_ = 1600000
G_ = 128
WK = 2000
ES = E_ // 16
NWIN = ES // WK


def _make_sc_scatter(D, CS, nchunks, accr):
    CS4 = CS * 4
    OUTR = nchunks * CS4
    SH = accr // 16
    NZ = SH // 128
    OH = CS4 // 16
    DB = CS4
    nck = nchunks // 2
    mesh = plsc.VectorSubcoreMesh(core_axis_name="c", subcore_axis_name="s")

    @functools.partial(
        pl.kernel,
        out_type=jax.ShapeDtypeStruct((OUTR, D), jnp.float32),
        mesh=mesh,
        compiler_params=pltpu.CompilerParams(needs_layout_passes=False,
                                             use_tc_tiling_on_sc=False),
        scratch_types=[
            pltpu.VMEM((WK,), jnp.int32),
            pltpu.VMEM((WK,), jnp.int32),
            pltpu.VMEM((WK + 128,), jnp.int32),
            pltpu.VMEM((WK + 128,), jnp.int32),
            pltpu.VMEM((16, 128), jnp.int32),
            pltpu.VMEM((128, D), jnp.float32),
            pltpu.VMEM((128, D), jnp.float32),
            pltpu.VMEM_SHARED((accr, D), jnp.float32),
            pltpu.SemaphoreType.DMA,
        ],
    )
    def k(table, keys, srcs, zrows, out,
          kwin, swin, stage_i, stage_s, cidx, upd, zbuf, acc, sem):
        cid = lax.axis_index("c")
        sid = lax.axis_index("s")
        lanes = lax.iota(jnp.int32, 16)
        pltpu.sync_copy(zrows, zbuf)

        def chunk_body(ci, carry):
            c = ci * 2 + cid
            base4 = c * CS4

            def zb(j, carry2):
                pltpu.sync_copy(zbuf, acc.at[pl.ds(sid * SH + j * 128, 128)])
                return carry2
            lax.fori_loop(0, NZ, zb, 0)
            plsc.subcore_barrier()

            def win(w, carry2):
                off = sid * ES + w * WK
                pltpu.sync_copy(keys.at[pl.ds(off, WK)], kwin)
                pltpu.sync_copy(srcs.at[pl.ds(off, WK)], swin)

                def vec(i, cur):
                    kv = kwin[pl.ds(i * 16, 16)]
                    lv = kv - base4
                    m = (lv >= 0) & (lv < CS4)
                    mi = jnp.where(m, 1, 0).astype(jnp.int32)
                    incl = jnp.cumsum(mi)
                    pos = cur + incl - mi
                    plsc.store_scatter(stage_i, [pos], lv, mask=m)
                    sv = swin[pl.ds(i * 16, 16)]
                    plsc.store_scatter(stage_s, [pos], sv, mask=m)
                    return cur + jnp.max(incl)
                cur = lax.fori_loop(0, WK // 16, vec, jnp.int32(0))

                for p in range(8):
                    stage_i[pl.ds(cur + p * 16, 16)] = DB + p * 16 + lanes
                    stage_s[pl.ds(cur + p * 16, 16)] = p * 16 + lanes
                nb = (cur + 127) // 128

                def fire(b, carry3):
                    for kk in range(8):
                        cidx[b, pl.ds(kk * 16, 16)] = (
                            stage_i[pl.ds(b * 128 + kk * 16, 16)])
                    pltpu.async_copy(
                        table.at[stage_s.at[pl.ds(b * 128, 128)]], upd, sem
                    ).wait()
                    pltpu.sync_copy(upd, acc.at[cidx.at[b]], add=True)
                    return carry3
                lax.fori_loop(0, nb, fire, 0)
                return carry2
            lax.fori_loop(0, NWIN, win, 0)
            plsc.subcore_barrier()
            pltpu.sync_copy(acc.at[pl.ds(sid * OH, OH)],
                            out.at[pl.ds(base4 + sid * OH, OH)])
            plsc.subcore_barrier()
            return carry
        lax.fori_loop(0, nck, chunk_body, 0)

    return k


_sc_layer1 = _make_sc_scatter(D=8, CS=25056, nchunks=4, accr=100352)
_sc_layer2 = _make_sc_scatter(D=80, CS=4576, nchunks=22, accr=18432)

_BN = 2000
_NBLK = N_ // _BN


def _tc1_body(s_ref, x_ref, w_ref, r1_ref, h_ref, ic_ref):
    s = s_ref[...]
    xb = x_ref[...]
    acc = jnp.zeros((_BN, 80), jnp.float32)
    ics = []
    for r in range(4):
        sr = s[:, r * 8:(r + 1) * 8]
        icr = 1.0 / jnp.maximum(sr[:, 6:7], 1.0)
        ics.append(icr)
        acc = acc + jnp.dot(sr * icr, w_ref[r],
                            preferred_element_type=jnp.float32)
    acc = acc + jnp.dot(xb, r1_ref[...], preferred_element_type=jnp.float32)
    h_ref[...] = jnp.maximum(acc, 0.0)
    ic_ref[...] = jnp.concatenate(ics, axis=1)


_tc1 = pl.pallas_call(
    _tc1_body,
    grid=(_NBLK,),
    in_specs=[
        pl.BlockSpec((_BN, 32), lambda i: (i, 0)),
        pl.BlockSpec((_BN, 8), lambda i: (i, 0)),
        pl.BlockSpec((4, 8, 80), lambda i: (0, 0, 0)),
        pl.BlockSpec((8, 80), lambda i: (0, 0)),
    ],
    out_specs=[
        pl.BlockSpec((_BN, 80), lambda i: (i, 0)),
        pl.BlockSpec((_BN, 4), lambda i: (i, 0)),
    ],
    out_shape=[
        jax.ShapeDtypeStruct((N_, 80), jnp.float32),
        jax.ShapeDtypeStruct((N_, 4), jnp.float32),
    ],
)


def _tc2_body(s_ref, ic_ref, h1_ref, b_ref, w2_ref, r2_ref, b2_ref,
              lw_ref, lb_ref, out_ref):
    i = pl.program_id(0)
    s = s_ref[...]
    ic = ic_ref[...]
    h1 = h1_ref[...]
    acc = jnp.zeros((_BN, 80), jnp.float32)
    for r in range(4):
        acc = acc + jnp.dot(s[:, r * 80:(r + 1) * 80] * ic[:, r:r + 1],
                            w2_ref[r], preferred_element_type=jnp.float32)
    h2 = jnp.maximum(
        acc + jnp.dot(h1, r2_ref[...], preferred_element_type=jnp.float32)
        + b2_ref[...], 0.0)
    y = jnp.dot(h2, lw_ref[...], preferred_element_type=jnp.float32)
    oneh = (b_ref[...] == lax.broadcasted_iota(jnp.int32, (1, G_), 1)
            ).astype(jnp.float32)
    part = lax.dot_general(oneh, y, (((0,), (0,)), ((), ())),
                           preferred_element_type=jnp.float32)

    @pl.when(i == 0)
    def _():
        out_ref[...] = jnp.broadcast_to(lb_ref[...], (G_, 15))
    out_ref[...] += part


_tc2 = pl.pallas_call(
    _tc2_body,
    grid=(_NBLK,),
    in_specs=[
        pl.BlockSpec((_BN, 320), lambda i: (i, 0)),
        pl.BlockSpec((_BN, 4), lambda i: (i, 0)),
        pl.BlockSpec((_BN, 80), lambda i: (i, 0)),
        pl.BlockSpec((_BN, 1), lambda i: (i, 0)),
        pl.BlockSpec((4, 80, 80), lambda i: (0, 0, 0)),
        pl.BlockSpec((80, 80), lambda i: (0, 0)),
        pl.BlockSpec((1, 80), lambda i: (0, 0)),
        pl.BlockSpec((80, 15), lambda i: (0, 0)),
        pl.BlockSpec((1, 15), lambda i: (0, 0)),
    ],
    out_specs=pl.BlockSpec((G_, 15), lambda i: (0, 0)),
    out_shape=jax.ShapeDtypeStruct((G_, 15), jnp.float32),
)


def kernel(x, edge_index, edge_type, batch,
           W1, root1, b1, W2, root2, b2, linW, linb):
    x_aug = jnp.concatenate(
        [x, jnp.ones((N_, 1), jnp.float32), jnp.zeros((N_, 1), jnp.float32)],
        axis=1)
    keys = edge_index[1] * 4 + edge_type
    srcs = edge_index[0]
    z8 = jnp.zeros((128, 8), jnp.float32)
    z80 = jnp.zeros((128, 80), jnp.float32)

    S1 = _sc_layer1(x_aug, keys, srcs, z8)
    W1p = jnp.pad(W1, ((0, 0), (0, 2), (0, 0)))
    root1p = jnp.concatenate(
        [root1, b1[None, :], jnp.zeros((1, 80), jnp.float32)], axis=0)
    h1, invc = _tc1(S1[:4 * N_].reshape(N_, 32), x_aug, W1p, root1p)

    S2 = _sc_layer2(h1, keys, srcs, z80)
    out = _tc2(S2[:4 * N_].reshape(N_, 320), invc, h1, batch[:, None],
               W2, root2, b2[None, :], linW, linb[None, :])
    return out


# speedup vs baseline: 5.7348x; 1.2755x over previous
import functools

import jax
import jax.numpy as jnp
from jax import lax
from jax.experimental import pallas as pl
from jax.experimental.pallas import tpu as pltpu
from jax.experimental.pallas import tpu_sc as plsc

N_ = 100000
E_ = 1600000
G_ = 128
WK = 2000
ES = E_ // 16
NWIN = ES // WK
UNR = 5


def _make_sc_scatter(D, CS, nchunks, accr):
    CS4 = CS * 4
    OUTR = nchunks * CS4
    SH = accr // 16
    NZ = SH // 128
    OH = CS4 // 16
    DB = CS4
    nck = nchunks // 2
    mesh = plsc.VectorSubcoreMesh(core_axis_name="c", subcore_axis_name="s")

    @functools.partial(
        pl.kernel,
        out_type=jax.ShapeDtypeStruct((OUTR, D), jnp.float32),
        mesh=mesh,
        compiler_params=pltpu.CompilerParams(needs_layout_passes=False,
                                             use_tc_tiling_on_sc=False),
        scratch_types=[
            pltpu.VMEM((WK,), jnp.int32),
            pltpu.VMEM((WK,), jnp.int32),
            pltpu.VMEM((WK + 256,), jnp.int32),
            pltpu.VMEM((WK + 256,), jnp.int32),
            pltpu.VMEM((16, 128), jnp.int32),
            pltpu.VMEM((128, D), jnp.float32),
            pltpu.VMEM_SHARED((accr, D), jnp.float32),
            pltpu.SemaphoreType.DMA,
        ],
    )
    def k(table, keys, srcs, zrows, out,
          kwin, swin, stage_i, stage_s, cidx, upd, acc, sem):
        cid = lax.axis_index("c")
        sid = lax.axis_index("s")
        lanes = lax.iota(jnp.int32, 16)

        def fire(b, carry3):
            ad = pltpu.async_copy(
                table.at[stage_s.at[pl.ds(b * 128, 128)]], upd, sem)
            for kk in range(8):
                cidx[b, pl.ds(kk * 16, 16)] = (
                    stage_i[pl.ds(b * 128 + kk * 16, 16)])
            ad.wait()
            pltpu.sync_copy(upd, acc.at[cidx.at[b]], add=True)
            return carry3

        def chunk_body(ci, carry):
            c = ci * 2 + cid
            base4 = c * CS4

            def zb(j, carry2):
                pltpu.sync_copy(zrows, acc.at[pl.ds(sid * SH + j * 128, 128)])
                return carry2
            lax.fori_loop(0, NZ, zb, 0)
            plsc.subcore_barrier()

            def win(w, cur):
                off = sid * ES + w * WK
                pltpu.sync_copy(keys.at[pl.ds(off, WK)], kwin)
                pltpu.sync_copy(srcs.at[pl.ds(off, WK)], swin)

                def vec(i, cur2):
                    incls = []
                    for u in range(UNR):
                        kv = kwin[pl.ds((i * UNR + u) * 16, 16)]
                        lv = kv - base4
                        m = (lv >= 0) & (lv < CS4)
                        mi = jnp.where(m, 1, 0).astype(jnp.int32)
                        sv = swin[pl.ds((i * UNR + u) * 16, 16)]
                        incls.append((jnp.cumsum(mi), mi, lv, sv, m))
                    for incl, mi, lv, sv, m in incls:
                        pos = cur2 + incl - mi
                        plsc.store_scatter(stage_i, [pos], lv, mask=m)
                        plsc.store_scatter(stage_s, [pos], sv, mask=m)
                        cur2 = cur2 + incl[15]
                    return cur2
                cur = lax.fori_loop(0, WK // (16 * UNR), vec, cur)

                nb = cur // 128
                lax.fori_loop(0, nb, fire, 0)
                for kk in range(8):
                    stage_i[pl.ds(kk * 16, 16)] = (
                        stage_i[pl.ds(nb * 128 + kk * 16, 16)])
                    stage_s[pl.ds(kk * 16, 16)] = (
                        stage_s[pl.ds(nb * 128 + kk * 16, 16)])
                return cur - nb * 128
            cur = lax.fori_loop(0, NWIN, win, jnp.int32(0))

            for p in range(8):
                stage_i[pl.ds(cur + p * 16, 16)] = DB + p * 16 + lanes
                stage_s[pl.ds(cur + p * 16, 16)] = p * 16 + lanes
            lax.fori_loop(0, (cur + 127) // 128, fire, 0)
            plsc.subcore_barrier()
            pltpu.sync_copy(acc.at[pl.ds(sid * OH, OH)],
                            out.at[pl.ds(base4 + sid * OH, OH)])
            plsc.subcore_barrier()
            return carry
        lax.fori_loop(0, nck, chunk_body, 0)

    return k


_sc_layer1 = _make_sc_scatter(D=8, CS=50048, nchunks=2, accr=200704)
_sc_layer2 = _make_sc_scatter(D=80, CS=5088, nchunks=20, accr=20480)

_BN = 2000
_NBLK = N_ // _BN


def _tc1_body(s_ref, x_ref, w_ref, r1_ref, h_ref, ic_ref):
    s = s_ref[...]
    xb = x_ref[...]
    acc = jnp.zeros((_BN, 80), jnp.float32)
    ics = []
    for r in range(4):
        sr = s[:, r * 8:(r + 1) * 8]
        icr = 1.0 / jnp.maximum(sr[:, 6:7], 1.0)
        ics.append(icr)
        acc = acc + jnp.dot(sr * icr, w_ref[r],
                            preferred_element_type=jnp.float32)
    acc = acc + jnp.dot(xb, r1_ref[...], preferred_element_type=jnp.float32)
    h_ref[...] = jnp.maximum(acc, 0.0)
    ic_ref[...] = jnp.concatenate(ics, axis=1)


_tc1 = pl.pallas_call(
    _tc1_body,
    grid=(_NBLK,),
    in_specs=[
        pl.BlockSpec((_BN, 32), lambda i: (i, 0)),
        pl.BlockSpec((_BN, 8), lambda i: (i, 0)),
        pl.BlockSpec((4, 8, 80), lambda i: (0, 0, 0)),
        pl.BlockSpec((8, 80), lambda i: (0, 0)),
    ],
    out_specs=[
        pl.BlockSpec((_BN, 80), lambda i: (i, 0)),
        pl.BlockSpec((_BN, 4), lambda i: (i, 0)),
    ],
    out_shape=[
        jax.ShapeDtypeStruct((N_, 80), jnp.float32),
        jax.ShapeDtypeStruct((N_, 4), jnp.float32),
    ],
)


def _tc2_body(s_ref, ic_ref, h1_ref, b_ref, w2_ref, r2_ref, b2_ref,
              lw_ref, lb_ref, out_ref):
    i = pl.program_id(0)
    s = s_ref[...]
    ic = ic_ref[...]
    h1 = h1_ref[...]
    acc = jnp.zeros((_BN, 80), jnp.float32)
    for r in range(4):
        acc = acc + jnp.dot(s[:, r * 80:(r + 1) * 80] * ic[:, r:r + 1],
                            w2_ref[r], preferred_element_type=jnp.float32)
    h2 = jnp.maximum(
        acc + jnp.dot(h1, r2_ref[...], preferred_element_type=jnp.float32)
        + b2_ref[...], 0.0)
    y = jnp.dot(h2, lw_ref[...], preferred_element_type=jnp.float32)
    oneh = (b_ref[...] == lax.broadcasted_iota(jnp.int32, (1, G_), 1)
            ).astype(jnp.float32)
    part = lax.dot_general(oneh, y, (((0,), (0,)), ((), ())),
                           preferred_element_type=jnp.float32)

    @pl.when(i == 0)
    def _():
        out_ref[...] = jnp.broadcast_to(lb_ref[...], (G_, 15))
    out_ref[...] += part


_tc2 = pl.pallas_call(
    _tc2_body,
    grid=(_NBLK,),
    in_specs=[
        pl.BlockSpec((_BN, 320), lambda i: (i, 0)),
        pl.BlockSpec((_BN, 4), lambda i: (i, 0)),
        pl.BlockSpec((_BN, 80), lambda i: (i, 0)),
        pl.BlockSpec((_BN, 1), lambda i: (i, 0)),
        pl.BlockSpec((4, 80, 80), lambda i: (0, 0, 0)),
        pl.BlockSpec((80, 80), lambda i: (0, 0)),
        pl.BlockSpec((1, 80), lambda i: (0, 0)),
        pl.BlockSpec((80, 15), lambda i: (0, 0)),
        pl.BlockSpec((1, 15), lambda i: (0, 0)),
    ],
    out_specs=pl.BlockSpec((G_, 15), lambda i: (0, 0)),
    out_shape=jax.ShapeDtypeStruct((G_, 15), jnp.float32),
)


def kernel(x, edge_index, edge_type, batch,
           W1, root1, b1, W2, root2, b2, linW, linb):
    x_aug = jnp.concatenate(
        [x, jnp.ones((N_, 1), jnp.float32), jnp.zeros((N_, 1), jnp.float32)],
        axis=1)
    keys = edge_index[1] * 4 + edge_type
    srcs = edge_index[0]
    z8 = jnp.zeros((128, 8), jnp.float32)
    z80 = jnp.zeros((128, 80), jnp.float32)

    S1 = _sc_layer1(x_aug, keys, srcs, z8)
    W1p = jnp.pad(W1, ((0, 0), (0, 2), (0, 0)))
    root1p = jnp.concatenate(
        [root1, b1[None, :], jnp.zeros((1, 80), jnp.float32)], axis=0)
    h1, invc = _tc1(S1[:4 * N_].reshape(N_, 32), x_aug, W1p, root1p)

    S2 = _sc_layer2(h1, keys, srcs, z80)
    out = _tc2(S2[:4 * N_].reshape(N_, 320), invc, h1, batch[:, None],
               W2, root2, b2[None, :], linW, linb[None, :])
    return out


# speedup vs baseline: 6.9444x; 1.2109x over previous
import functools

import jax
import jax.numpy as jnp
from jax import lax
from jax.experimental import pallas as pl
from jax.experimental.pallas import tpu as pltpu
from jax.experimental.pallas import tpu_sc as plsc

N_ = 100000
E_ = 1600000
G_ = 128
WK = 2000
ES = E_ // 16
NWIN = ES // WK
UNR = 5


def _make_sc_scatter(D, CS, nchunks, accr):
    CS4 = CS * 4
    OUTR = nchunks * CS4
    SH = accr // 16
    NZ = SH // 128
    OH = CS4 // 16
    DB = CS4
    nck = nchunks // 2
    mesh = plsc.VectorSubcoreMesh(core_axis_name="c", subcore_axis_name="s")

    @functools.partial(
        pl.kernel,
        out_type=jax.ShapeDtypeStruct((OUTR, D), jnp.float32),
        mesh=mesh,
        compiler_params=pltpu.CompilerParams(needs_layout_passes=False,
                                             use_tc_tiling_on_sc=False),
        scratch_types=[
            pltpu.VMEM((2, WK), jnp.int32),
            pltpu.VMEM((2, WK), jnp.int32),
            pltpu.VMEM((WK + 256,), jnp.int32),
            pltpu.VMEM((WK + 256,), jnp.int32),
            pltpu.VMEM((16, 128), jnp.int32),
            pltpu.VMEM((128, D), jnp.float32),
            pltpu.VMEM_SHARED((accr, D), jnp.float32),
            pltpu.SemaphoreType.DMA,
            pltpu.SemaphoreType.DMA,
        ],
    )
    def k(table, keys, srcs, zrows, out,
          kwin, swin, stage_i, stage_s, cidx, upd, acc, sem, msem):
        cid = lax.axis_index("c")
        sid = lax.axis_index("s")
        lanes = lax.iota(jnp.int32, 16)

        def meta_start(w, p):
            off = sid * ES + w * WK
            pltpu.async_copy(keys.at[pl.ds(off, WK)], kwin.at[p], msem)
            pltpu.async_copy(srcs.at[pl.ds(off, WK)], swin.at[p], msem)

        def meta_wait(p):
            pltpu.make_async_copy(keys.at[pl.ds(0, WK)], kwin.at[p],
                                  msem).wait()
            pltpu.make_async_copy(srcs.at[pl.ds(0, WK)], swin.at[p],
                                  msem).wait()

        def fire(b, carry3):
            ad = pltpu.async_copy(
                table.at[stage_s.at[pl.ds(b * 128, 128)]], upd, sem)
            for kk in range(8):
                cidx[b, pl.ds(kk * 16, 16)] = (
                    stage_i[pl.ds(b * 128 + kk * 16, 16)])
            ad.wait()
            pltpu.sync_copy(upd, acc.at[cidx.at[b]], add=True)
            return carry3

        def chunk_body(ci, carry):
            c = ci * 2 + cid
            base4 = c * CS4

            def zb(j, carry2):
                pltpu.sync_copy(zrows, acc.at[pl.ds(sid * SH + j * 128, 128)])
                return carry2
            meta_start(0, 0)
            lax.fori_loop(0, NZ, zb, 0)
            plsc.subcore_barrier()

            def win(w, cur):
                p = w & 1
                meta_wait(p)

                @pl.when(w + 1 < NWIN)
                def _():
                    meta_start(w + 1, 1 - p)

                def vec(i, cur2):
                    incls = []
                    for u in range(UNR):
                        kv = kwin[p, pl.ds((i * UNR + u) * 16, 16)]
                        lv = kv - base4
                        m = (lv >= 0) & (lv < CS4)
                        mi = jnp.where(m, 1, 0).astype(jnp.int32)
                        sv = swin[p, pl.ds((i * UNR + u) * 16, 16)]
                        incls.append((jnp.cumsum(mi), mi, lv, sv, m))
                    for incl, mi, lv, sv, m in incls:
                        pos = cur2 + incl - mi
                        plsc.store_scatter(stage_i, [pos], lv, mask=m)
                        plsc.store_scatter(stage_s, [pos], sv, mask=m)
                        cur2 = cur2 + incl[15]
                    return cur2
                cur = lax.fori_loop(0, WK // (16 * UNR), vec, cur)

                nb = cur // 128
                lax.fori_loop(0, nb, fire, 0)
                for kk in range(8):
                    stage_i[pl.ds(kk * 16, 16)] = (
                        stage_i[pl.ds(nb * 128 + kk * 16, 16)])
                    stage_s[pl.ds(kk * 16, 16)] = (
                        stage_s[pl.ds(nb * 128 + kk * 16, 16)])
                return cur - nb * 128
            cur = lax.fori_loop(0, NWIN, win, jnp.int32(0))

            for p in range(8):
                stage_i[pl.ds(cur + p * 16, 16)] = DB + p * 16 + lanes
                stage_s[pl.ds(cur + p * 16, 16)] = p * 16 + lanes
            lax.fori_loop(0, (cur + 127) // 128, fire, 0)
            plsc.subcore_barrier()
            pltpu.sync_copy(acc.at[pl.ds(sid * OH, OH)],
                            out.at[pl.ds(base4 + sid * OH, OH)])
            plsc.subcore_barrier()
            return carry
        lax.fori_loop(0, nck, chunk_body, 0)

    return k


_sc_layer1 = _make_sc_scatter(D=8, CS=50048, nchunks=2, accr=200704)
_sc_layer2 = _make_sc_scatter(D=80, CS=5088, nchunks=20, accr=20480)

_BN = 2000
_NBLK = N_ // _BN


def _tc1_body(s_ref, x_ref, w_ref, r1_ref, h_ref, ic_ref):
    s = s_ref[...]
    xb = x_ref[...]
    acc = jnp.zeros((_BN, 80), jnp.float32)
    ics = []
    for r in range(4):
        sr = s[:, r * 8:(r + 1) * 8]
        icr = 1.0 / jnp.maximum(sr[:, 6:7], 1.0)
        ics.append(icr)
        acc = acc + jnp.dot(sr * icr, w_ref[r],
                            preferred_element_type=jnp.float32)
    acc = acc + jnp.dot(xb, r1_ref[...], preferred_element_type=jnp.float32)
    h_ref[...] = jnp.maximum(acc, 0.0)
    ic_ref[...] = jnp.concatenate(ics, axis=1)


_tc1 = pl.pallas_call(
    _tc1_body,
    grid=(_NBLK,),
    in_specs=[
        pl.BlockSpec((_BN, 32), lambda i: (i, 0)),
        pl.BlockSpec((_BN, 8), lambda i: (i, 0)),
        pl.BlockSpec((4, 8, 80), lambda i: (0, 0, 0)),
        pl.BlockSpec((8, 80), lambda i: (0, 0)),
    ],
    out_specs=[
        pl.BlockSpec((_BN, 80), lambda i: (i, 0)),
        pl.BlockSpec((_BN, 4), lambda i: (i, 0)),
    ],
    out_shape=[
        jax.ShapeDtypeStruct((N_, 80), jnp.float32),
        jax.ShapeDtypeStruct((N_, 4), jnp.float32),
    ],
)


def _tc2_body(s_ref, ic_ref, h1_ref, b_ref, w2_ref, r2_ref, b2_ref,
              lw_ref, lb_ref, out_ref):
    i = pl.program_id(0)
    s = s_ref[...]
    ic = ic_ref[...]
    h1 = h1_ref[...]
    acc = jnp.zeros((_BN, 80), jnp.float32)
    for r in range(4):
        acc = acc + jnp.dot(s[:, r * 80:(r + 1) * 80] * ic[:, r:r + 1],
                            w2_ref[r], preferred_element_type=jnp.float32)
    h2 = jnp.maximum(
        acc + jnp.dot(h1, r2_ref[...], preferred_element_type=jnp.float32)
        + b2_ref[...], 0.0)
    y = jnp.dot(h2, lw_ref[...], preferred_element_type=jnp.float32)
    oneh = (b_ref[...] == lax.broadcasted_iota(jnp.int32, (1, G_), 1)
            ).astype(jnp.float32)
    part = lax.dot_general(oneh, y, (((0,), (0,)), ((), ())),
                           preferred_element_type=jnp.float32)

    @pl.when(i == 0)
    def _():
        out_ref[...] = jnp.broadcast_to(lb_ref[...], (G_, 15))
    out_ref[...] += part


_tc2 = pl.pallas_call(
    _tc2_body,
    grid=(_NBLK,),
    in_specs=[
        pl.BlockSpec((_BN, 320), lambda i: (i, 0)),
        pl.BlockSpec((_BN, 4), lambda i: (i, 0)),
        pl.BlockSpec((_BN, 80), lambda i: (i, 0)),
        pl.BlockSpec((_BN, 1), lambda i: (i, 0)),
        pl.BlockSpec((4, 80, 80), lambda i: (0, 0, 0)),
        pl.BlockSpec((80, 80), lambda i: (0, 0)),
        pl.BlockSpec((1, 80), lambda i: (0, 0)),
        pl.BlockSpec((80, 15), lambda i: (0, 0)),
        pl.BlockSpec((1, 15), lambda i: (0, 0)),
    ],
    out_specs=pl.BlockSpec((G_, 15), lambda i: (0, 0)),
    out_shape=jax.ShapeDtypeStruct((G_, 15), jnp.float32),
)


def kernel(x, edge_index, edge_type, batch,
           W1, root1, b1, W2, root2, b2, linW, linb):
    x_aug = jnp.concatenate(
        [x, jnp.ones((N_, 1), jnp.float32), jnp.zeros((N_, 1), jnp.float32)],
        axis=1)
    keys = edge_index[1] * 4 + edge_type
    srcs = edge_index[0]
    z8 = jnp.zeros((128, 8), jnp.float32)
    z80 = jnp.zeros((128, 80), jnp.float32)

    S1 = _sc_layer1(x_aug, keys, srcs, z8)
    W1p = jnp.pad(W1, ((0, 0), (0, 2), (0, 0)))
    root1p = jnp.concatenate(
        [root1, b1[None, :], jnp.zeros((1, 80), jnp.float32)], axis=0)
    h1, invc = _tc1(S1[:4 * N_].reshape(N_, 32), x_aug, W1p, root1p)

    S2 = _sc_layer2(h1, keys, srcs, z80)
    out = _tc2(S2[:4 * N_].reshape(N_, 320), invc, h1, batch[:, None],
               W2, root2, b2[None, :], linW, linb[None, :])
    return out
